# baseline (speedup 1.0000x reference)
import functools

import jax
import jax.numpy as jnp
from jax import lax
from jax.experimental import pallas as pl
from jax.experimental.pallas import tpu as pltpu
from jax.experimental.pallas import tpu_sc as plsc

_LRELU_SLOPE = 0.2
_BN_EPS = 1e-5
_W_EPS = 1e-8

_NC = 2
_NS = 16
_L = 16
_NW = _NC * _NS
_CHUNK = 64


def _fold128(x):
    acc = x[:, 0:128]
    for off in range(128, x.shape[1], 128):
        acc = acc + x[:, off:off + 128]
    return acc


def _knn_idx_body(x1_ref, x2_ref, idx_ref, w_ref, *, b_base):
    b = pl.program_id(0) + b_base
    x1 = x1_ref[0]
    x2 = x2_ref[0]
    s = x2.shape[1]
    tn = x1.shape[1]

    x1sq = jnp.sum(x1 * x1, axis=0)
    x2sq = jnp.sum(x2 * x2, axis=0)
    cross = jax.lax.dot_general(x2, x1, (((0,), (0,)), ((), ())),
                                preferred_element_type=jnp.float32)
    dist = x2sq[:, None] + x1sq[None, :] - 2.0 * cross

    big = jnp.float32(jnp.inf)
    m0 = jnp.min(dist, axis=0)
    eq0 = dist == m0[None, :]
    d1m = jnp.where(eq0, big, dist)
    m1 = jnp.min(d1m, axis=0)
    eq1 = d1m == m1[None, :]
    d2m = jnp.where(eq1, big, d1m)
    m2 = jnp.min(d2m, axis=0)
    eq2 = d2m == m2[None, :]

    iota_s = jax.lax.broadcasted_iota(jnp.int32, (s, tn), 0)
    sbig = jnp.int32(s)
    base = b * s
    i0 = jnp.min(jnp.where(eq0, iota_s, sbig), axis=0) + base
    i1 = jnp.min(jnp.where(eq1, iota_s, sbig), axis=0) + base
    i2 = jnp.min(jnp.where(eq2, iota_s, sbig), axis=0) + base

    r0 = 1.0 / (m0 + _W_EPS)
    r1 = 1.0 / (m1 + _W_EPS)
    r2 = 1.0 / (m2 + _W_EPS)
    inv_norm = 1.0 / (r0 + r1 + r2)
    idx_ref[0:1, :] = i0[None, :]
    idx_ref[1:2, :] = i1[None, :]
    idx_ref[2:3, :] = i2[None, :]
    w_ref[0:1, :] = (r0 * inv_norm)[None, :]
    w_ref[1:2, :] = (r1 * inv_norm)[None, :]
    w_ref[2:3, :] = (r2 * inv_norm)[None, :]


def _sc_gather_body(p2_hbm, i0_hbm, i1_hbm, i2_hbm, w0_hbm, w1_hbm, w2_hbm,
                    out_hbm, i0_v, i1_v, i2_v, w0_v, w1_v, w2_v,
                    r0_v, r1_v, r2_v, semg0, semg1, sems, *, bn, d):
    wid = lax.axis_index("s") * _NC + lax.axis_index("c")
    per_w = bn // _NW
    nch = per_w // _CHUNK
    wbase = wid * per_w

    pltpu.sync_copy(i0_hbm.at[pl.ds(wbase, per_w)], i0_v)
    pltpu.sync_copy(i1_hbm.at[pl.ds(wbase, per_w)], i1_v)
    pltpu.sync_copy(i2_hbm.at[pl.ds(wbase, per_w)], i2_v)
    pltpu.sync_copy(w0_hbm.at[pl.ds(wbase, per_w)], w0_v)
    pltpu.sync_copy(w1_hbm.at[pl.ds(wbase, per_w)], w1_v)
    pltpu.sync_copy(w2_hbm.at[pl.ds(wbase, per_w)], w2_v)

    def chunk(g, carry):
        coff = g * _CHUNK
        cp0 = pltpu.async_copy(p2_hbm.at[i0_v.at[pl.ds(coff, _CHUNK)]],
                               r0_v, semg0)
        cp1 = pltpu.async_copy(p2_hbm.at[i1_v.at[pl.ds(coff, _CHUNK)]],
                               r1_v, semg1)
        cp2 = pltpu.async_copy(p2_hbm.at[i2_v.at[pl.ds(coff, _CHUNK)]],
                               r2_v, sems)
        cp0.wait()
        cp1.wait()
        cp2.wait()

        def q(gg, c):
            wv0 = w0_v[pl.ds(coff + gg * _L, _L)]
            wv1 = w1_v[pl.ds(coff + gg * _L, _L)]
            wv2 = w2_v[pl.ds(coff + gg * _L, _L)]
            for e in range(_L):
                r = gg * _L + e
                w0s = wv0[e]
                w1s = wv1[e]
                w2s = wv2[e]
                for j in range(d // _L):
                    sl = pl.ds(j * _L, _L)
                    r0_v[r, sl] = (w0s * r0_v[r, sl]
                                   + w1s * r1_v[r, sl]
                                   + w2s * r2_v[r, sl])
            return c

        lax.fori_loop(0, _CHUNK // _L, q, 0)
        pltpu.sync_copy(r0_v, out_hbm.at[pl.ds(wbase + coff, _CHUNK)])
        return carry

    lax.fori_loop(0, nch, chunk, 0)


def _conv1_body(p1_ref, interp_ref, w1a_ref, w1b_ref, b1_ref, h1_ref, st1_ref):
    b = pl.program_id(0)
    t = pl.program_id(1)
    h1 = (jax.lax.dot_general(w1a_ref[...], p1_ref[0], (((1,), (0,)), ((), ())),
                              preferred_element_type=jnp.float32)
          + jax.lax.dot_general(w1b_ref[...], interp_ref[...],
                                (((1,), (1,)), ((), ())),
                                preferred_element_type=jnp.float32)
          + b1_ref[...])
    h1_ref[0] = h1

    psum = _fold128(h1)
    psq = _fold128(h1 * h1)

    @pl.when(jnp.logical_and(b == 0, t == 0))
    def _():
        st1_ref[0] = psum
        st1_ref[1] = psq

    @pl.when(jnp.logical_not(jnp.logical_and(b == 0, t == 0)))
    def _():
        st1_ref[0] += psum
        st1_ref[1] += psq


def _bn1_conv2_body(h1_ref, st1_ref, g1_ref, bb1_ref, w2_ref, b2_ref,
                    h2_ref, st2_ref, *, count, parts):
    b = pl.program_id(0)
    t = pl.program_id(1)
    inv = jnp.float32(1.0 / count)
    mean = jnp.sum(jnp.sum(st1_ref[0:parts], axis=0),
                   axis=1, keepdims=True) * inv
    ex2 = jnp.sum(jnp.sum(st1_ref[parts:2 * parts], axis=0),
                  axis=1, keepdims=True) * inv
    var = ex2 - mean * mean
    scale = g1_ref[...] / jnp.sqrt(var + _BN_EPS)
    shift = bb1_ref[...] - scale * mean
    a = scale * h1_ref[0] + shift
    a = jnp.where(a >= 0, a, _LRELU_SLOPE * a)
    h2 = jax.lax.dot_general(w2_ref[...], a, (((1,), (0,)), ((), ())),
                             preferred_element_type=jnp.float32) + b2_ref[...]
    h2_ref[0] = h2

    psum = _fold128(h2)
    psq = _fold128(h2 * h2)

    @pl.when(jnp.logical_and(b == 0, t == 0))
    def _():
        st2_ref[0] = psum
        st2_ref[1] = psq

    @pl.when(jnp.logical_not(jnp.logical_and(b == 0, t == 0)))
    def _():
        st2_ref[0] += psum
        st2_ref[1] += psq


def _bn2_body(h2_ref, st2_ref, g2_ref, bb2_ref, out_ref, *, count, parts):
    inv = jnp.float32(1.0 / count)
    mean = jnp.sum(jnp.sum(st2_ref[0:parts], axis=0),
                   axis=1, keepdims=True) * inv
    ex2 = jnp.sum(jnp.sum(st2_ref[parts:2 * parts], axis=0),
                  axis=1, keepdims=True) * inv
    var = ex2 - mean * mean
    scale = g2_ref[...] / jnp.sqrt(var + _BN_EPS)
    shift = bb2_ref[...] - scale * mean
    a = scale * h2_ref[0] + shift
    out_ref[0] = jnp.where(a >= 0, a, _LRELU_SLOPE * a)


def kernel(xyz1, xyz2, points1, points2, conv1_w, conv1_b, bn1_g, bn1_b,
           conv2_w, conv2_b, bn2_g, bn2_b):
    b, _, n = xyz1.shape
    s = xyz2.shape[2]
    d = points1.shape[1]
    c1 = conv1_w.shape[0]
    c2 = conv2_w.shape[0]
    tn = min(512, n)
    n_tiles = n // tn
    count = b * n
    bn = b * n

    w1a = conv1_w[:, :d]
    w1b = conv1_w[:, d:]
    b1 = conv1_b.reshape(c1, 1)
    g1 = bn1_g.reshape(c1, 1)
    bb1 = bn1_b.reshape(c1, 1)
    b2 = conv2_b.reshape(c2, 1)
    g2 = bn2_g.reshape(c2, 1)
    bb2 = bn2_b.reshape(c2, 1)

    parts = 4
    bh = b // parts
    bn_h = bh * n
    grid_h = (bh, n_tiles)

    p2flat = jnp.transpose(points2, (0, 2, 1)).reshape(b * s, d)

    mesh = plsc.VectorSubcoreMesh(core_axis_name="c", subcore_axis_name="s",
                                  num_cores=_NC, num_subcores=_NS)

    interps = []
    for h in range(parts):
        bsl = slice(h * bh, (h + 1) * bh)
        idx3, wgt3 = pl.pallas_call(
            functools.partial(_knn_idx_body, b_base=h * bh),
            grid=grid_h,
            in_specs=[
                pl.BlockSpec((1, 3, tn), lambda i, j: (i, 0, j)),
                pl.BlockSpec((1, 3, s), lambda i, j: (i, 0, 0)),
            ],
            out_specs=[
                pl.BlockSpec((3, tn), lambda i, j: (0, i * n_tiles + j)),
                pl.BlockSpec((3, tn), lambda i, j: (0, i * n_tiles + j)),
            ],
            out_shape=[
                jax.ShapeDtypeStruct((3, bn_h), jnp.int32),
                jax.ShapeDtypeStruct((3, bn_h), jnp.float32),
            ],
        )(xyz1[bsl], xyz2[bsl])

        interp = pl.kernel(
            functools.partial(_sc_gather_body, bn=bn_h, d=d),
            out_type=jax.ShapeDtypeStruct((bn_h, d), jnp.float32),
            mesh=mesh,
            scratch_types=[
                pltpu.VMEM((bn_h // _NW,), jnp.int32),
                pltpu.VMEM((bn_h // _NW,), jnp.int32),
                pltpu.VMEM((bn_h // _NW,), jnp.int32),
                pltpu.VMEM((bn_h // _NW,), jnp.float32),
                pltpu.VMEM((bn_h // _NW,), jnp.float32),
                pltpu.VMEM((bn_h // _NW,), jnp.float32),
                pltpu.VMEM((_CHUNK, d), jnp.float32),
                pltpu.VMEM((_CHUNK, d), jnp.float32),
                pltpu.VMEM((_CHUNK, d), jnp.float32),
                pltpu.SemaphoreType.DMA,
                pltpu.SemaphoreType.DMA,
                pltpu.SemaphoreType.DMA,
            ],
        )(p2flat, idx3[0], idx3[1], idx3[2], wgt3[0], wgt3[1], wgt3[2])
        interps.append(interp)

    h1s, st1s = [], []
    for h in range(parts):
        bsl = slice(h * bh, (h + 1) * bh)
        h1, st1 = pl.pallas_call(
            _conv1_body,
            grid=grid_h,
            in_specs=[
                pl.BlockSpec((1, d, tn), lambda i, j: (i, 0, j)),
                pl.BlockSpec((tn, d), lambda i, j: (i * n_tiles + j, 0)),
                pl.BlockSpec((c1, d), lambda i, j: (0, 0)),
                pl.BlockSpec((c1, d), lambda i, j: (0, 0)),
                pl.BlockSpec((c1, 1), lambda i, j: (0, 0)),
            ],
            out_specs=[
                pl.BlockSpec((1, c1, tn), lambda i, j: (i, 0, j)),
                pl.BlockSpec((2, c1, 128), lambda i, j: (0, 0, 0)),
            ],
            out_shape=[
                jax.ShapeDtypeStruct((bh, c1, n), jnp.float32),
                jax.ShapeDtypeStruct((2, c1, 128), jnp.float32),
            ],
        )(points1[bsl], interps[h], w1a, w1b, b1)
        h1s.append(h1)
        st1s.append(st1)

    st1cat = jnp.concatenate([st[0:1] for st in st1s]
                             + [st[1:2] for st in st1s], axis=0)

    h2s, st2s = [], []
    for h in range(parts):
        h2, st2 = pl.pallas_call(
            functools.partial(_bn1_conv2_body, count=count, parts=parts),
            grid=grid_h,
            in_specs=[
                pl.BlockSpec((1, c1, tn), lambda i, j: (i, 0, j)),
                pl.BlockSpec((2 * parts, c1, 128), lambda i, j: (0, 0, 0)),
                pl.BlockSpec((c1, 1), lambda i, j: (0, 0)),
                pl.BlockSpec((c1, 1), lambda i, j: (0, 0)),
                pl.BlockSpec((c2, c1), lambda i, j: (0, 0)),
                pl.BlockSpec((c2, 1), lambda i, j: (0, 0)),
            ],
            out_specs=[
                pl.BlockSpec((1, c2, tn), lambda i, j: (i, 0, j)),
                pl.BlockSpec((2, c2, 128), lambda i, j: (0, 0, 0)),
            ],
            out_shape=[
                jax.ShapeDtypeStruct((bh, c2, n), jnp.float32),
                jax.ShapeDtypeStruct((2, c2, 128), jnp.float32),
            ],
        )(h1s[h], st1cat, g1, bb1, conv2_w, b2)
        h2s.append(h2)
        st2s.append(st2)

    st2cat = jnp.concatenate([st[0:1] for st in st2s]
                             + [st[1:2] for st in st2s], axis=0)

    outs = []
    for h in range(parts):
        out_h = pl.pallas_call(
            functools.partial(_bn2_body, count=count, parts=parts),
            grid=grid_h,
            in_specs=[
                pl.BlockSpec((1, c2, tn), lambda i, j: (i, 0, j)),
                pl.BlockSpec((2 * parts, c2, 128), lambda i, j: (0, 0, 0)),
                pl.BlockSpec((c2, 1), lambda i, j: (0, 0)),
                pl.BlockSpec((c2, 1), lambda i, j: (0, 0)),
            ],
            out_specs=pl.BlockSpec((1, c2, tn), lambda i, j: (i, 0, j)),
            out_shape=jax.ShapeDtypeStruct((bh, c2, n), jnp.float32),
        )(h2s[h], st2cat, g2, bb2)
        outs.append(out_h)

    return jnp.concatenate(outs, axis=0)


# speedup vs baseline: 1.0350x; 1.0350x over previous
import functools

import jax
import jax.numpy as jnp
from jax import lax
from jax.experimental import pallas as pl
from jax.experimental.pallas import tpu as pltpu
from jax.experimental.pallas import tpu_sc as plsc

_LRELU_SLOPE = 0.2
_BN_EPS = 1e-5
_W_EPS = 1e-8

_NC = 2
_NS = 16
_L = 16
_NW = _NC * _NS
_CHUNK = 64


def _fold128(x):
    acc = x[:, 0:128]
    for off in range(128, x.shape[1], 128):
        acc = acc + x[:, off:off + 128]
    return acc


def _knn_idx_body(x1_ref, x2_ref, idx_ref, w_ref, *, b_base):
    b = pl.program_id(0) + b_base
    x1 = x1_ref[0]
    x2 = x2_ref[0]
    s = x2.shape[1]
    tn = x1.shape[1]

    x1sq = jnp.sum(x1 * x1, axis=0)
    x2sq = jnp.sum(x2 * x2, axis=0)
    cross = jax.lax.dot_general(x2, x1, (((0,), (0,)), ((), ())),
                                preferred_element_type=jnp.float32)
    dist = x2sq[:, None] + x1sq[None, :] - 2.0 * cross

    big = jnp.float32(jnp.inf)
    m0 = jnp.min(dist, axis=0)
    eq0 = dist == m0[None, :]
    d1m = jnp.where(eq0, big, dist)
    m1 = jnp.min(d1m, axis=0)
    eq1 = d1m == m1[None, :]
    d2m = jnp.where(eq1, big, d1m)
    m2 = jnp.min(d2m, axis=0)
    eq2 = d2m == m2[None, :]

    iota_s = jax.lax.broadcasted_iota(jnp.int32, (s, tn), 0)
    sbig = jnp.int32(s)
    base = b * s
    i0 = jnp.min(jnp.where(eq0, iota_s, sbig), axis=0) + base
    i1 = jnp.min(jnp.where(eq1, iota_s, sbig), axis=0) + base
    i2 = jnp.min(jnp.where(eq2, iota_s, sbig), axis=0) + base

    r0 = 1.0 / (m0 + _W_EPS)
    r1 = 1.0 / (m1 + _W_EPS)
    r2 = 1.0 / (m2 + _W_EPS)
    inv_norm = 1.0 / (r0 + r1 + r2)
    idx_ref[0:1, :] = i0[None, :]
    idx_ref[1:2, :] = i1[None, :]
    idx_ref[2:3, :] = i2[None, :]
    w_ref[0:1, :] = (r0 * inv_norm)[None, :]
    w_ref[1:2, :] = (r1 * inv_norm)[None, :]
    w_ref[2:3, :] = (r2 * inv_norm)[None, :]


def _sc_gather_body(p2_hbm, i0_hbm, i1_hbm, i2_hbm, w0_hbm, w1_hbm, w2_hbm,
                    out_hbm, i0_v, i1_v, i2_v, w0_v, w1_v, w2_v,
                    r0_v, r1_v, r2_v, semg0, semg1, sems, *, bn, d):
    wid = lax.axis_index("s") * _NC + lax.axis_index("c")
    per_w = bn // _NW
    nch = per_w // _CHUNK
    wbase = wid * per_w

    pltpu.sync_copy(i0_hbm.at[pl.ds(wbase, per_w)], i0_v)
    pltpu.sync_copy(i1_hbm.at[pl.ds(wbase, per_w)], i1_v)
    pltpu.sync_copy(i2_hbm.at[pl.ds(wbase, per_w)], i2_v)
    pltpu.sync_copy(w0_hbm.at[pl.ds(wbase, per_w)], w0_v)
    pltpu.sync_copy(w1_hbm.at[pl.ds(wbase, per_w)], w1_v)
    pltpu.sync_copy(w2_hbm.at[pl.ds(wbase, per_w)], w2_v)

    def chunk(g, carry):
        coff = g * _CHUNK
        cp0 = pltpu.async_copy(p2_hbm.at[i0_v.at[pl.ds(coff, _CHUNK)]],
                               r0_v, semg0)
        cp1 = pltpu.async_copy(p2_hbm.at[i1_v.at[pl.ds(coff, _CHUNK)]],
                               r1_v, semg1)
        cp2 = pltpu.async_copy(p2_hbm.at[i2_v.at[pl.ds(coff, _CHUNK)]],
                               r2_v, sems)
        cp0.wait()
        cp1.wait()
        cp2.wait()

        def q(gg, c):
            wv0 = w0_v[pl.ds(coff + gg * _L, _L)]
            wv1 = w1_v[pl.ds(coff + gg * _L, _L)]
            wv2 = w2_v[pl.ds(coff + gg * _L, _L)]
            for e in range(_L):
                r = gg * _L + e
                w0s = wv0[e]
                w1s = wv1[e]
                w2s = wv2[e]
                for j in range(d // _L):
                    sl = pl.ds(j * _L, _L)
                    r0_v[r, sl] = (w0s * r0_v[r, sl]
                                   + w1s * r1_v[r, sl]
                                   + w2s * r2_v[r, sl])
            return c

        lax.fori_loop(0, _CHUNK // _L, q, 0)
        pltpu.sync_copy(r0_v, out_hbm.at[pl.ds(wbase + coff, _CHUNK)])
        return carry

    lax.fori_loop(0, nch, chunk, 0)


def _conv1_body(p1_ref, interp_ref, w1a_ref, w1b_ref, b1_ref, h1_ref, st1_ref):
    b = pl.program_id(0)
    t = pl.program_id(1)
    h1 = (jax.lax.dot_general(w1a_ref[...], p1_ref[0], (((1,), (0,)), ((), ())),
                              preferred_element_type=jnp.float32)
          + jax.lax.dot_general(w1b_ref[...], interp_ref[...],
                                (((1,), (1,)), ((), ())),
                                preferred_element_type=jnp.float32)
          + b1_ref[...])
    h1_ref[0] = h1

    psum = _fold128(h1)
    psq = _fold128(h1 * h1)

    @pl.when(jnp.logical_and(b == 0, t == 0))
    def _():
        st1_ref[0] = psum
        st1_ref[1] = psq

    @pl.when(jnp.logical_not(jnp.logical_and(b == 0, t == 0)))
    def _():
        st1_ref[0] += psum
        st1_ref[1] += psq


def _bn1_conv2_body(h1_ref, st1_ref, g1_ref, bb1_ref, w2_ref, b2_ref,
                    h2_ref, st2_ref, *, count, parts):
    b = pl.program_id(0)
    t = pl.program_id(1)
    inv = jnp.float32(1.0 / count)
    mean = jnp.sum(jnp.sum(st1_ref[0:parts], axis=0),
                   axis=1, keepdims=True) * inv
    ex2 = jnp.sum(jnp.sum(st1_ref[parts:2 * parts], axis=0),
                  axis=1, keepdims=True) * inv
    var = ex2 - mean * mean
    scale = g1_ref[...] / jnp.sqrt(var + _BN_EPS)
    shift = bb1_ref[...] - scale * mean
    a = scale * h1_ref[0] + shift
    a = jnp.where(a >= 0, a, _LRELU_SLOPE * a)
    h2 = jax.lax.dot_general(w2_ref[...], a, (((1,), (0,)), ((), ())),
                             preferred_element_type=jnp.float32) + b2_ref[...]
    h2_ref[0] = h2

    psum = _fold128(h2)
    psq = _fold128(h2 * h2)

    @pl.when(jnp.logical_and(b == 0, t == 0))
    def _():
        st2_ref[0] = psum
        st2_ref[1] = psq

    @pl.when(jnp.logical_not(jnp.logical_and(b == 0, t == 0)))
    def _():
        st2_ref[0] += psum
        st2_ref[1] += psq


def _bn2_body(h2_ref, st2_ref, g2_ref, bb2_ref, out_ref, *, count, parts):
    inv = jnp.float32(1.0 / count)
    mean = jnp.sum(jnp.sum(st2_ref[0:parts], axis=0),
                   axis=1, keepdims=True) * inv
    ex2 = jnp.sum(jnp.sum(st2_ref[parts:2 * parts], axis=0),
                  axis=1, keepdims=True) * inv
    var = ex2 - mean * mean
    scale = g2_ref[...] / jnp.sqrt(var + _BN_EPS)
    shift = bb2_ref[...] - scale * mean
    a = scale * h2_ref[0] + shift
    out_ref[0] = jnp.where(a >= 0, a, _LRELU_SLOPE * a)


def kernel(xyz1, xyz2, points1, points2, conv1_w, conv1_b, bn1_g, bn1_b,
           conv2_w, conv2_b, bn2_g, bn2_b):
    b, _, n = xyz1.shape
    s = xyz2.shape[2]
    d = points1.shape[1]
    c1 = conv1_w.shape[0]
    c2 = conv2_w.shape[0]
    tn = min(512, n)
    n_tiles = n // tn
    count = b * n
    bn = b * n

    w1a = conv1_w[:, :d]
    w1b = conv1_w[:, d:]
    b1 = conv1_b.reshape(c1, 1)
    g1 = bn1_g.reshape(c1, 1)
    bb1 = bn1_b.reshape(c1, 1)
    b2 = conv2_b.reshape(c2, 1)
    g2 = bn2_g.reshape(c2, 1)
    bb2 = bn2_b.reshape(c2, 1)

    parts = 2
    bh = b // parts
    bn_h = bh * n
    grid_h = (bh, n_tiles)

    p2flat = jnp.transpose(points2, (0, 2, 1)).reshape(b * s, d)

    mesh = plsc.VectorSubcoreMesh(core_axis_name="c", subcore_axis_name="s",
                                  num_cores=_NC, num_subcores=_NS)

    interps = []
    for h in range(parts):
        bsl = slice(h * bh, (h + 1) * bh)
        idx3, wgt3 = pl.pallas_call(
            functools.partial(_knn_idx_body, b_base=h * bh),
            grid=grid_h,
            in_specs=[
                pl.BlockSpec((1, 3, tn), lambda i, j: (i, 0, j)),
                pl.BlockSpec((1, 3, s), lambda i, j: (i, 0, 0)),
            ],
            out_specs=[
                pl.BlockSpec((3, tn), lambda i, j: (0, i * n_tiles + j)),
                pl.BlockSpec((3, tn), lambda i, j: (0, i * n_tiles + j)),
            ],
            out_shape=[
                jax.ShapeDtypeStruct((3, bn_h), jnp.int32),
                jax.ShapeDtypeStruct((3, bn_h), jnp.float32),
            ],
        )(xyz1[bsl], xyz2[bsl])

        interp = pl.kernel(
            functools.partial(_sc_gather_body, bn=bn_h, d=d),
            out_type=jax.ShapeDtypeStruct((bn_h, d), jnp.float32),
            mesh=mesh,
            scratch_types=[
                pltpu.VMEM((bn_h // _NW,), jnp.int32),
                pltpu.VMEM((bn_h // _NW,), jnp.int32),
                pltpu.VMEM((bn_h // _NW,), jnp.int32),
                pltpu.VMEM((bn_h // _NW,), jnp.float32),
                pltpu.VMEM((bn_h // _NW,), jnp.float32),
                pltpu.VMEM((bn_h // _NW,), jnp.float32),
                pltpu.VMEM((_CHUNK, d), jnp.float32),
                pltpu.VMEM((_CHUNK, d), jnp.float32),
                pltpu.VMEM((_CHUNK, d), jnp.float32),
                pltpu.SemaphoreType.DMA,
                pltpu.SemaphoreType.DMA,
                pltpu.SemaphoreType.DMA,
            ],
        )(p2flat, idx3[0], idx3[1], idx3[2], wgt3[0], wgt3[1], wgt3[2])
        interps.append(interp)

    h1s, st1s = [], []
    for h in range(parts):
        bsl = slice(h * bh, (h + 1) * bh)
        h1, st1 = pl.pallas_call(
            _conv1_body,
            grid=grid_h,
            in_specs=[
                pl.BlockSpec((1, d, tn), lambda i, j: (i, 0, j)),
                pl.BlockSpec((tn, d), lambda i, j: (i * n_tiles + j, 0)),
                pl.BlockSpec((c1, d), lambda i, j: (0, 0)),
                pl.BlockSpec((c1, d), lambda i, j: (0, 0)),
                pl.BlockSpec((c1, 1), lambda i, j: (0, 0)),
            ],
            out_specs=[
                pl.BlockSpec((1, c1, tn), lambda i, j: (i, 0, j)),
                pl.BlockSpec((2, c1, 128), lambda i, j: (0, 0, 0)),
            ],
            out_shape=[
                jax.ShapeDtypeStruct((bh, c1, n), jnp.float32),
                jax.ShapeDtypeStruct((2, c1, 128), jnp.float32),
            ],
        )(points1[bsl], interps[h], w1a, w1b, b1)
        h1s.append(h1)
        st1s.append(st1)

    st1cat = jnp.concatenate([st[0:1] for st in st1s]
                             + [st[1:2] for st in st1s], axis=0)

    h2s, st2s = [], []
    for h in range(parts):
        h2, st2 = pl.pallas_call(
            functools.partial(_bn1_conv2_body, count=count, parts=parts),
            grid=grid_h,
            in_specs=[
                pl.BlockSpec((1, c1, tn), lambda i, j: (i, 0, j)),
                pl.BlockSpec((2 * parts, c1, 128), lambda i, j: (0, 0, 0)),
                pl.BlockSpec((c1, 1), lambda i, j: (0, 0)),
                pl.BlockSpec((c1, 1), lambda i, j: (0, 0)),
                pl.BlockSpec((c2, c1), lambda i, j: (0, 0)),
                pl.BlockSpec((c2, 1), lambda i, j: (0, 0)),
            ],
            out_specs=[
                pl.BlockSpec((1, c2, tn), lambda i, j: (i, 0, j)),
                pl.BlockSpec((2, c2, 128), lambda i, j: (0, 0, 0)),
            ],
            out_shape=[
                jax.ShapeDtypeStruct((bh, c2, n), jnp.float32),
                jax.ShapeDtypeStruct((2, c2, 128), jnp.float32),
            ],
        )(h1s[h], st1cat, g1, bb1, conv2_w, b2)
        h2s.append(h2)
        st2s.append(st2)

    st2cat = jnp.concatenate([st[0:1] for st in st2s]
                             + [st[1:2] for st in st2s], axis=0)

    outs = []
    for h in range(parts):
        out_h = pl.pallas_call(
            functools.partial(_bn2_body, count=count, parts=parts),
            grid=grid_h,
            in_specs=[
                pl.BlockSpec((1, c2, tn), lambda i, j: (i, 0, j)),
                pl.BlockSpec((2 * parts, c2, 128), lambda i, j: (0, 0, 0)),
                pl.BlockSpec((c2, 1), lambda i, j: (0, 0)),
                pl.BlockSpec((c2, 1), lambda i, j: (0, 0)),
            ],
            out_specs=pl.BlockSpec((1, c2, tn), lambda i, j: (i, 0, j)),
            out_shape=jax.ShapeDtypeStruct((bh, c2, n), jnp.float32),
        )(h2s[h], st2cat, g2, bb2)
        outs.append(out_h)

    return jnp.concatenate(outs, axis=0)


# speedup vs baseline: 1.0491x; 1.0136x over previous
import functools

import jax
import jax.numpy as jnp
from jax import lax
from jax.experimental import pallas as pl
from jax.experimental.pallas import tpu as pltpu
from jax.experimental.pallas import tpu_sc as plsc

_LRELU_SLOPE = 0.2
_BN_EPS = 1e-5
_W_EPS = 1e-8

_NC = 2
_NS = 16
_L = 16
_NW = _NC * _NS
_CHUNK = 64


def _fold128(x):
    acc = x[:, 0:128]
    for off in range(128, x.shape[1], 128):
        acc = acc + x[:, off:off + 128]
    return acc


def _knn_idx_body(x1_ref, x2_ref, idx_ref, w_ref, *, b_base):
    b = pl.program_id(0) + b_base
    x1 = x1_ref[0]
    x2 = x2_ref[0]
    s = x2.shape[1]
    tn = x1.shape[1]

    x1sq = jnp.sum(x1 * x1, axis=0)
    x2sq = jnp.sum(x2 * x2, axis=0)
    cross = jax.lax.dot_general(x2, x1, (((0,), (0,)), ((), ())),
                                preferred_element_type=jnp.float32)
    dist = x2sq[:, None] + x1sq[None, :] - 2.0 * cross

    big = jnp.float32(jnp.inf)
    m0 = jnp.min(dist, axis=0)
    eq0 = dist == m0[None, :]
    d1m = jnp.where(eq0, big, dist)
    m1 = jnp.min(d1m, axis=0)
    eq1 = d1m == m1[None, :]
    d2m = jnp.where(eq1, big, d1m)
    m2 = jnp.min(d2m, axis=0)
    eq2 = d2m == m2[None, :]

    iota_s = jax.lax.broadcasted_iota(jnp.int32, (s, tn), 0)
    sbig = jnp.int32(s)
    base = b * s
    i0 = jnp.min(jnp.where(eq0, iota_s, sbig), axis=0) + base
    i1 = jnp.min(jnp.where(eq1, iota_s, sbig), axis=0) + base
    i2 = jnp.min(jnp.where(eq2, iota_s, sbig), axis=0) + base

    r0 = 1.0 / (m0 + _W_EPS)
    r1 = 1.0 / (m1 + _W_EPS)
    r2 = 1.0 / (m2 + _W_EPS)
    inv_norm = 1.0 / (r0 + r1 + r2)
    idx_ref[0:1, :] = i0[None, :]
    idx_ref[1:2, :] = i1[None, :]
    idx_ref[2:3, :] = i2[None, :]
    w_ref[0:1, :] = (r0 * inv_norm)[None, :]
    w_ref[1:2, :] = (r1 * inv_norm)[None, :]
    w_ref[2:3, :] = (r2 * inv_norm)[None, :]


def _sc_gather_body(p2_hbm, i0_hbm, i1_hbm, i2_hbm, w0_hbm, w1_hbm, w2_hbm,
                    out_hbm, i0_v, i1_v, i2_v, w0_v, w1_v, w2_v,
                    r0_v, r1_v, r2_v, semg0, semg1, sems, *, bn, d):
    wid = lax.axis_index("s") * _NC + lax.axis_index("c")
    per_w = bn // _NW
    nch = per_w // _CHUNK
    wbase = wid * per_w

    pltpu.sync_copy(i0_hbm.at[pl.ds(wbase, per_w)], i0_v)
    pltpu.sync_copy(i1_hbm.at[pl.ds(wbase, per_w)], i1_v)
    pltpu.sync_copy(i2_hbm.at[pl.ds(wbase, per_w)], i2_v)
    pltpu.sync_copy(w0_hbm.at[pl.ds(wbase, per_w)], w0_v)
    pltpu.sync_copy(w1_hbm.at[pl.ds(wbase, per_w)], w1_v)
    pltpu.sync_copy(w2_hbm.at[pl.ds(wbase, per_w)], w2_v)

    def chunk(g, carry):
        coff = g * _CHUNK
        cp0 = pltpu.async_copy(p2_hbm.at[i0_v.at[pl.ds(coff, _CHUNK)]],
                               r0_v, semg0)
        cp1 = pltpu.async_copy(p2_hbm.at[i1_v.at[pl.ds(coff, _CHUNK)]],
                               r1_v, semg1)
        cp2 = pltpu.async_copy(p2_hbm.at[i2_v.at[pl.ds(coff, _CHUNK)]],
                               r2_v, sems)
        cp0.wait()
        cp1.wait()
        cp2.wait()

        def q(gg, c):
            wv0 = w0_v[pl.ds(coff + gg * _L, _L)]
            wv1 = w1_v[pl.ds(coff + gg * _L, _L)]
            wv2 = w2_v[pl.ds(coff + gg * _L, _L)]
            for e in range(_L):
                r = gg * _L + e
                w0s = wv0[e]
                w1s = wv1[e]
                w2s = wv2[e]
                for j in range(d // _L):
                    sl = pl.ds(j * _L, _L)
                    r0_v[r, sl] = (w0s * r0_v[r, sl]
                                   + w1s * r1_v[r, sl]
                                   + w2s * r2_v[r, sl])
            return c

        lax.fori_loop(0, _CHUNK // _L, q, 0)
        pltpu.sync_copy(r0_v, out_hbm.at[pl.ds(wbase + coff, _CHUNK)])
        return carry

    lax.fori_loop(0, nch, chunk, 0)


def _conv1_body(p1_ref, interp_ref, w1a_ref, w1b_ref, b1_ref, h1_ref, st1_ref):
    b = pl.program_id(0)
    t = pl.program_id(1)
    h1 = (jax.lax.dot_general(w1a_ref[...], p1_ref[0], (((1,), (0,)), ((), ())),
                              preferred_element_type=jnp.float32)
          + jax.lax.dot_general(w1b_ref[...], interp_ref[...],
                                (((1,), (1,)), ((), ())),
                                preferred_element_type=jnp.float32)
          + b1_ref[...])
    h1_ref[0] = h1

    psum = _fold128(h1)
    psq = _fold128(h1 * h1)

    @pl.when(jnp.logical_and(b == 0, t == 0))
    def _():
        st1_ref[0] = psum
        st1_ref[1] = psq

    @pl.when(jnp.logical_not(jnp.logical_and(b == 0, t == 0)))
    def _():
        st1_ref[0] += psum
        st1_ref[1] += psq


def _bn1_conv2_body(h1_ref, st1a_ref, st1b_ref, g1_ref, bb1_ref, w2_ref, b2_ref,
                    h2_ref, st2_ref, *, count):
    b = pl.program_id(0)
    t = pl.program_id(1)
    inv = jnp.float32(1.0 / count)
    mean = jnp.sum(st1a_ref[0] + st1b_ref[0], axis=1, keepdims=True) * inv
    ex2 = jnp.sum(st1a_ref[1] + st1b_ref[1], axis=1, keepdims=True) * inv
    var = ex2 - mean * mean
    scale = g1_ref[...] / jnp.sqrt(var + _BN_EPS)
    shift = bb1_ref[...] - scale * mean
    a = scale * h1_ref[0] + shift
    a = jnp.where(a >= 0, a, _LRELU_SLOPE * a)
    h2 = jax.lax.dot_general(w2_ref[...], a, (((1,), (0,)), ((), ())),
                             preferred_element_type=jnp.float32) + b2_ref[...]
    h2_ref[0] = h2

    psum = _fold128(h2)
    psq = _fold128(h2 * h2)

    @pl.when(jnp.logical_and(b == 0, t == 0))
    def _():
        st2_ref[0] = psum
        st2_ref[1] = psq

    @pl.when(jnp.logical_not(jnp.logical_and(b == 0, t == 0)))
    def _():
        st2_ref[0] += psum
        st2_ref[1] += psq


def _bn2_body(h2_ref, st2a_ref, st2b_ref, g2_ref, bb2_ref, out_ref, *, count):
    inv = jnp.float32(1.0 / count)
    mean = jnp.sum(st2a_ref[0] + st2b_ref[0], axis=1, keepdims=True) * inv
    ex2 = jnp.sum(st2a_ref[1] + st2b_ref[1], axis=1, keepdims=True) * inv
    var = ex2 - mean * mean
    scale = g2_ref[...] / jnp.sqrt(var + _BN_EPS)
    shift = bb2_ref[...] - scale * mean
    a = scale * h2_ref[0] + shift
    out_ref[0] = jnp.where(a >= 0, a, _LRELU_SLOPE * a)


def kernel(xyz1, xyz2, points1, points2, conv1_w, conv1_b, bn1_g, bn1_b,
           conv2_w, conv2_b, bn2_g, bn2_b):
    b, _, n = xyz1.shape
    s = xyz2.shape[2]
    d = points1.shape[1]
    c1 = conv1_w.shape[0]
    c2 = conv2_w.shape[0]
    tn = min(512, n)
    n_tiles = n // tn
    count = b * n
    bn = b * n

    w1a = conv1_w[:, :d]
    w1b = conv1_w[:, d:]
    b1 = conv1_b.reshape(c1, 1)
    g1 = bn1_g.reshape(c1, 1)
    bb1 = bn1_b.reshape(c1, 1)
    b2 = conv2_b.reshape(c2, 1)
    g2 = bn2_g.reshape(c2, 1)
    bb2 = bn2_b.reshape(c2, 1)

    parts = 2
    bh = b // parts
    bn_h = bh * n
    grid_h = (bh, n_tiles)

    p2flat = jnp.transpose(points2, (0, 2, 1)).reshape(b * s, d)

    mesh = plsc.VectorSubcoreMesh(core_axis_name="c", subcore_axis_name="s",
                                  num_cores=_NC, num_subcores=_NS)

    interps = []
    for h in range(parts):
        bsl = slice(h * bh, (h + 1) * bh)
        idx3, wgt3 = pl.pallas_call(
            functools.partial(_knn_idx_body, b_base=h * bh),
            grid=grid_h,
            in_specs=[
                pl.BlockSpec((1, 3, tn), lambda i, j: (i, 0, j)),
                pl.BlockSpec((1, 3, s), lambda i, j: (i, 0, 0)),
            ],
            out_specs=[
                pl.BlockSpec((3, tn), lambda i, j: (0, i * n_tiles + j)),
                pl.BlockSpec((3, tn), lambda i, j: (0, i * n_tiles + j)),
            ],
            out_shape=[
                jax.ShapeDtypeStruct((3, bn_h), jnp.int32),
                jax.ShapeDtypeStruct((3, bn_h), jnp.float32),
            ],
        )(xyz1[bsl], xyz2[bsl])

        interp = pl.kernel(
            functools.partial(_sc_gather_body, bn=bn_h, d=d),
            out_type=jax.ShapeDtypeStruct((bn_h, d), jnp.float32),
            mesh=mesh,
            scratch_types=[
                pltpu.VMEM((bn_h // _NW,), jnp.int32),
                pltpu.VMEM((bn_h // _NW,), jnp.int32),
                pltpu.VMEM((bn_h // _NW,), jnp.int32),
                pltpu.VMEM((bn_h // _NW,), jnp.float32),
                pltpu.VMEM((bn_h // _NW,), jnp.float32),
                pltpu.VMEM((bn_h // _NW,), jnp.float32),
                pltpu.VMEM((_CHUNK, d), jnp.float32),
                pltpu.VMEM((_CHUNK, d), jnp.float32),
                pltpu.VMEM((_CHUNK, d), jnp.float32),
                pltpu.SemaphoreType.DMA,
                pltpu.SemaphoreType.DMA,
                pltpu.SemaphoreType.DMA,
            ],
        )(p2flat, idx3[0], idx3[1], idx3[2], wgt3[0], wgt3[1], wgt3[2])
        interps.append(interp)

    h1s, st1s = [], []
    for h in range(parts):
        bsl = slice(h * bh, (h + 1) * bh)
        h1, st1 = pl.pallas_call(
            _conv1_body,
            grid=grid_h,
            in_specs=[
                pl.BlockSpec((1, d, tn), lambda i, j: (i, 0, j)),
                pl.BlockSpec((tn, d), lambda i, j: (i * n_tiles + j, 0)),
                pl.BlockSpec((c1, d), lambda i, j: (0, 0)),
                pl.BlockSpec((c1, d), lambda i, j: (0, 0)),
                pl.BlockSpec((c1, 1), lambda i, j: (0, 0)),
            ],
            out_specs=[
                pl.BlockSpec((1, c1, tn), lambda i, j: (i, 0, j)),
                pl.BlockSpec((2, c1, 128), lambda i, j: (0, 0, 0)),
            ],
            out_shape=[
                jax.ShapeDtypeStruct((bh, c1, n), jnp.float32),
                jax.ShapeDtypeStruct((2, c1, 128), jnp.float32),
            ],
        )(points1[bsl], interps[h], w1a, w1b, b1)
        h1s.append(h1)
        st1s.append(st1)

    h2s, st2s = [], []
    for h in range(parts):
        h2, st2 = pl.pallas_call(
            functools.partial(_bn1_conv2_body, count=count),
            grid=grid_h,
            in_specs=[
                pl.BlockSpec((1, c1, tn), lambda i, j: (i, 0, j)),
                pl.BlockSpec((2, c1, 128), lambda i, j: (0, 0, 0)),
                pl.BlockSpec((2, c1, 128), lambda i, j: (0, 0, 0)),
                pl.BlockSpec((c1, 1), lambda i, j: (0, 0)),
                pl.BlockSpec((c1, 1), lambda i, j: (0, 0)),
                pl.BlockSpec((c2, c1), lambda i, j: (0, 0)),
                pl.BlockSpec((c2, 1), lambda i, j: (0, 0)),
            ],
            out_specs=[
                pl.BlockSpec((1, c2, tn), lambda i, j: (i, 0, j)),
                pl.BlockSpec((2, c2, 128), lambda i, j: (0, 0, 0)),
            ],
            out_shape=[
                jax.ShapeDtypeStruct((bh, c2, n), jnp.float32),
                jax.ShapeDtypeStruct((2, c2, 128), jnp.float32),
            ],
        )(h1s[h], st1s[0], st1s[1], g1, bb1, conv2_w, b2)
        h2s.append(h2)
        st2s.append(st2)

    outs = []
    for h in range(parts):
        out_h = pl.pallas_call(
            functools.partial(_bn2_body, count=count),
            grid=grid_h,
            in_specs=[
                pl.BlockSpec((1, c2, tn), lambda i, j: (i, 0, j)),
                pl.BlockSpec((2, c2, 128), lambda i, j: (0, 0, 0)),
                pl.BlockSpec((2, c2, 128), lambda i, j: (0, 0, 0)),
                pl.BlockSpec((c2, 1), lambda i, j: (0, 0)),
                pl.BlockSpec((c2, 1), lambda i, j: (0, 0)),
            ],
            out_specs=pl.BlockSpec((1, c2, tn), lambda i, j: (i, 0, j)),
            out_shape=jax.ShapeDtypeStruct((bh, c2, n), jnp.float32),
        )(h2s[h], st2s[0], st2s[1], g2, bb2)
        outs.append(out_h)

    return jnp.concatenate(outs, axis=0)


# speedup vs baseline: 1.0502x; 1.0011x over previous
import functools

import jax
import jax.numpy as jnp
from jax import lax
from jax.experimental import pallas as pl
from jax.experimental.pallas import tpu as pltpu
from jax.experimental.pallas import tpu_sc as plsc

_LRELU_SLOPE = 0.2
_BN_EPS = 1e-5
_W_EPS = 1e-8

_NC = 2
_NS = 16
_L = 16
_NW = _NC * _NS
_CHUNK = 64


def _fold128(x):
    acc = x[:, 0:128]
    for off in range(128, x.shape[1], 128):
        acc = acc + x[:, off:off + 128]
    return acc


def _knn_idx_body(x1_ref, x2_ref, idx_ref, w_ref, *, b_base):
    b = pl.program_id(0) + b_base
    x1 = x1_ref[0]
    x2 = x2_ref[0]
    s = x2.shape[1]
    tn = x1.shape[1]

    x1sq = jnp.sum(x1 * x1, axis=0)
    x2sq = jnp.sum(x2 * x2, axis=0)
    cross = jax.lax.dot_general(x2, x1, (((0,), (0,)), ((), ())),
                                preferred_element_type=jnp.float32)
    dist = x2sq[:, None] + x1sq[None, :] - 2.0 * cross

    big = jnp.float32(jnp.inf)
    m0 = jnp.min(dist, axis=0)
    eq0 = dist == m0[None, :]
    d1m = jnp.where(eq0, big, dist)
    m1 = jnp.min(d1m, axis=0)
    eq1 = d1m == m1[None, :]
    d2m = jnp.where(eq1, big, d1m)
    m2 = jnp.min(d2m, axis=0)
    eq2 = d2m == m2[None, :]

    iota_s = jax.lax.broadcasted_iota(jnp.int32, (s, tn), 0)
    sbig = jnp.int32(s)
    base = b * s
    i0 = jnp.min(jnp.where(eq0, iota_s, sbig), axis=0) + base
    i1 = jnp.min(jnp.where(eq1, iota_s, sbig), axis=0) + base
    i2 = jnp.min(jnp.where(eq2, iota_s, sbig), axis=0) + base

    r0 = 1.0 / (m0 + _W_EPS)
    r1 = 1.0 / (m1 + _W_EPS)
    r2 = 1.0 / (m2 + _W_EPS)
    inv_norm = 1.0 / (r0 + r1 + r2)
    idx_ref[0:1, :] = i0[None, :]
    idx_ref[1:2, :] = i1[None, :]
    idx_ref[2:3, :] = i2[None, :]
    w_ref[0:1, :] = (r0 * inv_norm)[None, :]
    w_ref[1:2, :] = (r1 * inv_norm)[None, :]
    w_ref[2:3, :] = (r2 * inv_norm)[None, :]


def _sc_gather_body(p2_hbm, i0_hbm, i1_hbm, i2_hbm, w0_hbm, w1_hbm, w2_hbm,
                    out_hbm, i0_v, i1_v, i2_v, w0_v, w1_v, w2_v,
                    r0_v, r1_v, r2_v, semg0, semg1, sems, *, bn, d):
    wid = lax.axis_index("s") * _NC + lax.axis_index("c")
    per_w = bn // _NW
    nch = per_w // _CHUNK
    wbase = wid * per_w

    pltpu.sync_copy(i0_hbm.at[pl.ds(wbase, per_w)], i0_v)
    pltpu.sync_copy(i1_hbm.at[pl.ds(wbase, per_w)], i1_v)
    pltpu.sync_copy(i2_hbm.at[pl.ds(wbase, per_w)], i2_v)
    pltpu.sync_copy(w0_hbm.at[pl.ds(wbase, per_w)], w0_v)
    pltpu.sync_copy(w1_hbm.at[pl.ds(wbase, per_w)], w1_v)
    pltpu.sync_copy(w2_hbm.at[pl.ds(wbase, per_w)], w2_v)

    def chunk(g, carry):
        coff = g * _CHUNK
        cp0 = pltpu.async_copy(p2_hbm.at[i0_v.at[pl.ds(coff, _CHUNK)]],
                               r0_v, semg0)
        cp1 = pltpu.async_copy(p2_hbm.at[i1_v.at[pl.ds(coff, _CHUNK)]],
                               r1_v, semg1)
        cp2 = pltpu.async_copy(p2_hbm.at[i2_v.at[pl.ds(coff, _CHUNK)]],
                               r2_v, sems)
        cp0.wait()
        cp1.wait()
        cp2.wait()

        def q(gg, c):
            wv0 = w0_v[pl.ds(coff + gg * _L, _L)]
            wv1 = w1_v[pl.ds(coff + gg * _L, _L)]
            wv2 = w2_v[pl.ds(coff + gg * _L, _L)]
            for e in range(_L):
                r = gg * _L + e
                w0s = wv0[e]
                w1s = wv1[e]
                w2s = wv2[e]
                for j in range(d // _L):
                    sl = pl.ds(j * _L, _L)
                    r0_v[r, sl] = (w0s * r0_v[r, sl]
                                   + w1s * r1_v[r, sl]
                                   + w2s * r2_v[r, sl])
            return c

        lax.fori_loop(0, _CHUNK // _L, q, 0)
        pltpu.sync_copy(r0_v, out_hbm.at[pl.ds(wbase + coff, _CHUNK)])
        return carry

    lax.fori_loop(0, nch, chunk, 0)


def _conv1_body(p1_ref, interp_ref, w1a_ref, w1b_ref, b1_ref, h1_ref, st1_ref):
    b = pl.program_id(0)
    t = pl.program_id(1)
    h1 = (jax.lax.dot_general(w1a_ref[...], p1_ref[0], (((1,), (0,)), ((), ())),
                              preferred_element_type=jnp.float32)
          + jax.lax.dot_general(w1b_ref[...], interp_ref[...],
                                (((1,), (1,)), ((), ())),
                                preferred_element_type=jnp.float32)
          + b1_ref[...])
    h1_ref[0] = h1

    psum = _fold128(h1)
    psq = _fold128(h1 * h1)

    @pl.when(jnp.logical_and(b == 0, t == 0))
    def _():
        st1_ref[0] = psum
        st1_ref[1] = psq

    @pl.when(jnp.logical_not(jnp.logical_and(b == 0, t == 0)))
    def _():
        st1_ref[0] += psum
        st1_ref[1] += psq


def _bn1_conv2_body(h1_ref, st1a_ref, st1b_ref, g1_ref, bb1_ref, w2_ref, b2_ref,
                    h2_ref, st2_ref, *, count):
    b = pl.program_id(0)
    t = pl.program_id(1)
    inv = jnp.float32(1.0 / count)
    mean = jnp.sum(st1a_ref[0] + st1b_ref[0], axis=1, keepdims=True) * inv
    ex2 = jnp.sum(st1a_ref[1] + st1b_ref[1], axis=1, keepdims=True) * inv
    var = ex2 - mean * mean
    scale = g1_ref[...] / jnp.sqrt(var + _BN_EPS)
    shift = bb1_ref[...] - scale * mean
    a = scale * h1_ref[0] + shift
    a = jnp.where(a >= 0, a, _LRELU_SLOPE * a)
    h2 = jax.lax.dot_general(w2_ref[...], a, (((1,), (0,)), ((), ())),
                             preferred_element_type=jnp.float32) + b2_ref[...]
    h2_ref[0] = h2

    psum = _fold128(h2)
    psq = _fold128(h2 * h2)

    @pl.when(jnp.logical_and(b == 0, t == 0))
    def _():
        st2_ref[0] = psum
        st2_ref[1] = psq

    @pl.when(jnp.logical_not(jnp.logical_and(b == 0, t == 0)))
    def _():
        st2_ref[0] += psum
        st2_ref[1] += psq


def _bn2_body(h2_ref, st2a_ref, st2b_ref, g2_ref, bb2_ref, out_ref, *, count):
    inv = jnp.float32(1.0 / count)
    mean = jnp.sum(st2a_ref[0] + st2b_ref[0], axis=1, keepdims=True) * inv
    ex2 = jnp.sum(st2a_ref[1] + st2b_ref[1], axis=1, keepdims=True) * inv
    var = ex2 - mean * mean
    scale = g2_ref[...] / jnp.sqrt(var + _BN_EPS)
    shift = bb2_ref[...] - scale * mean
    a = scale * h2_ref[0] + shift
    out_ref[0] = jnp.where(a >= 0, a, _LRELU_SLOPE * a)


def kernel(xyz1, xyz2, points1, points2, conv1_w, conv1_b, bn1_g, bn1_b,
           conv2_w, conv2_b, bn2_g, bn2_b):
    b, _, n = xyz1.shape
    s = xyz2.shape[2]
    d = points1.shape[1]
    c1 = conv1_w.shape[0]
    c2 = conv2_w.shape[0]
    tn = min(512, n)
    n_tiles = n // tn
    count = b * n

    w1a = conv1_w[:, :d]
    w1b = conv1_w[:, d:]
    b1 = conv1_b.reshape(c1, 1)
    g1 = bn1_g.reshape(c1, 1)
    bb1 = bn1_b.reshape(c1, 1)
    b2 = conv2_b.reshape(c2, 1)
    g2 = bn2_g.reshape(c2, 1)
    bb2 = bn2_b.reshape(c2, 1)

    parts = 2
    bh = b // parts
    bn_h = bh * n
    grid_h = (bh, n_tiles)

    p2flat = jnp.transpose(points2, (0, 2, 1)).reshape(b * s, d)

    mesh = plsc.VectorSubcoreMesh(core_axis_name="c", subcore_axis_name="s",
                                  num_cores=_NC, num_subcores=_NS)

    interps = []
    for h in range(parts):
        bsl = slice(h * bh, (h + 1) * bh)
        idx3, wgt3 = pl.pallas_call(
            functools.partial(_knn_idx_body, b_base=h * bh),
            grid=grid_h,
            in_specs=[
                pl.BlockSpec((1, 3, tn), lambda i, j: (i, 0, j)),
                pl.BlockSpec((1, 3, s), lambda i, j: (i, 0, 0)),
            ],
            out_specs=[
                pl.BlockSpec((3, tn), lambda i, j: (0, i * n_tiles + j)),
                pl.BlockSpec((3, tn), lambda i, j: (0, i * n_tiles + j)),
            ],
            out_shape=[
                jax.ShapeDtypeStruct((3, bn_h), jnp.int32),
                jax.ShapeDtypeStruct((3, bn_h), jnp.float32),
            ],
        )(xyz1[bsl], xyz2[bsl])

        interp = pl.kernel(
            functools.partial(_sc_gather_body, bn=bn_h, d=d),
            out_type=jax.ShapeDtypeStruct((bn_h, d), jnp.float32),
            mesh=mesh,
            scratch_types=[
                pltpu.VMEM((bn_h // _NW,), jnp.int32),
                pltpu.VMEM((bn_h // _NW,), jnp.int32),
                pltpu.VMEM((bn_h // _NW,), jnp.int32),
                pltpu.VMEM((bn_h // _NW,), jnp.float32),
                pltpu.VMEM((bn_h // _NW,), jnp.float32),
                pltpu.VMEM((bn_h // _NW,), jnp.float32),
                pltpu.VMEM((_CHUNK, d), jnp.float32),
                pltpu.VMEM((_CHUNK, d), jnp.float32),
                pltpu.VMEM((_CHUNK, d), jnp.float32),
                pltpu.SemaphoreType.DMA,
                pltpu.SemaphoreType.DMA,
                pltpu.SemaphoreType.DMA,
            ],
        )(p2flat, idx3[0], idx3[1], idx3[2], wgt3[0], wgt3[1], wgt3[2])
        interps.append(interp)

    h1s, st1s = [], []
    for h in range(parts):
        bsl = slice(h * bh, (h + 1) * bh)
        h1, st1 = pl.pallas_call(
            _conv1_body,
            grid=grid_h,
            in_specs=[
                pl.BlockSpec((1, d, tn), lambda i, j: (i, 0, j)),
                pl.BlockSpec((tn, d), lambda i, j: (i * n_tiles + j, 0)),
                pl.BlockSpec((c1, d), lambda i, j: (0, 0)),
                pl.BlockSpec((c1, d), lambda i, j: (0, 0)),
                pl.BlockSpec((c1, 1), lambda i, j: (0, 0)),
            ],
            out_specs=[
                pl.BlockSpec((1, c1, tn), lambda i, j: (i, 0, j)),
                pl.BlockSpec((2, c1, 128), lambda i, j: (0, 0, 0)),
            ],
            out_shape=[
                jax.ShapeDtypeStruct((bh, c1, n), jnp.float32),
                jax.ShapeDtypeStruct((2, c1, 128), jnp.float32),
            ],
        )(points1[bsl], interps[h], w1a, w1b, b1)
        h1s.append(h1)
        st1s.append(st1)

    h2s, st2s = [], []
    for h in range(parts):
        h2, st2 = pl.pallas_call(
            functools.partial(_bn1_conv2_body, count=count),
            grid=grid_h,
            in_specs=[
                pl.BlockSpec((1, c1, tn), lambda i, j: (i, 0, j)),
                pl.BlockSpec((2, c1, 128), lambda i, j: (0, 0, 0)),
                pl.BlockSpec((2, c1, 128), lambda i, j: (0, 0, 0)),
                pl.BlockSpec((c1, 1), lambda i, j: (0, 0)),
                pl.BlockSpec((c1, 1), lambda i, j: (0, 0)),
                pl.BlockSpec((c2, c1), lambda i, j: (0, 0)),
                pl.BlockSpec((c2, 1), lambda i, j: (0, 0)),
            ],
            out_specs=[
                pl.BlockSpec((1, c2, tn), lambda i, j: (i, 0, j)),
                pl.BlockSpec((2, c2, 128), lambda i, j: (0, 0, 0)),
            ],
            out_shape=[
                jax.ShapeDtypeStruct((bh, c2, n), jnp.float32),
                jax.ShapeDtypeStruct((2, c2, 128), jnp.float32),
            ],
        )(h1s[h], st1s[0], st1s[1], g1, bb1, conv2_w, b2)
        h2s.append(h2)
        st2s.append(st2)

    outs = []
    for h in range(parts):
        out_h = pl.pallas_call(
            functools.partial(_bn2_body, count=count),
            grid=grid_h,
            in_specs=[
                pl.BlockSpec((1, c2, tn), lambda i, j: (i, 0, j)),
                pl.BlockSpec((2, c2, 128), lambda i, j: (0, 0, 0)),
                pl.BlockSpec((2, c2, 128), lambda i, j: (0, 0, 0)),
                pl.BlockSpec((c2, 1), lambda i, j: (0, 0)),
                pl.BlockSpec((c2, 1), lambda i, j: (0, 0)),
            ],
            out_specs=pl.BlockSpec((1, c2, tn), lambda i, j: (i, 0, j)),
            out_shape=jax.ShapeDtypeStruct((bh, c2, n), jnp.float32),
        )(h2s[h], st2s[0], st2s[1], g2, bb2)
        outs.append(out_h)

    return jnp.concatenate(outs, axis=0)
